# baseline (speedup 1.0000x reference)
import functools

import jax
import jax.numpy as jnp
from jax import lax
from jax.experimental import pallas as pl
from jax.experimental.pallas import tpu as pltpu
from jax.experimental.pallas import tpu_sc as plsc

_NC = 2
_NS = 16
_NW = _NC * _NS


def kernel(x, table):
    B, length, _ = x.shape
    V, D = table.shape
    rows_per_w = length // _NW
    CHUNK = 64
    n_chunks = rows_per_w // CHUNK

    mesh = plsc.VectorSubcoreMesh(core_axis_name="c", subcore_axis_name="s")

    @functools.partial(
        pl.kernel,
        out_type=jax.ShapeDtypeStruct((B, length, D), table.dtype),
        mesh=mesh,
        scratch_types=[pltpu.SemaphoreType.DMA],
    )
    def sc_copy(table_hbm, out_hbm, sem):
        wid = lax.axis_index("s") * _NC + lax.axis_index("c")
        base = wid * rows_per_w
        src = table_hbm.at[pl.ds(base, rows_per_w)]
        handles = [
            pltpu.async_copy(src, out_hbm.at[b, pl.ds(base, rows_per_w)], sem)
            for b in range(B)
        ]
        for h in handles:
            h.wait()

    return sc_copy(table)


def _copy_body(t_ref, o_ref):
    o_ref[0] = t_ref[...]


def _kernel_tc(x, table):
    B, length, _ = x.shape
    _, D = table.shape
    BLK = 512
    out = pl.pallas_call(
        _copy_body,
        grid=(length // BLK, B),
        in_specs=[pl.BlockSpec((BLK, D), lambda i, b: (i, 0))],
        out_specs=pl.BlockSpec((1, BLK, D), lambda i, b: (b, i, 0)),
        out_shape=jax.ShapeDtypeStruct((B, length, D), table.dtype),
    )(table)
    return out


# speedup vs baseline: 49.8353x; 49.8353x over previous
import functools

import jax
import jax.numpy as jnp
from jax import lax
from jax.experimental import pallas as pl
from jax.experimental.pallas import tpu as pltpu
from jax.experimental.pallas import tpu_sc as plsc

_NC = 2
_NS = 16
_NW = _NC * _NS


def kernel(x, table):
    B, length, _ = x.shape
    V, D = table.shape
    rows_per_w = length // _NW
    CHUNK = 32
    n_chunks = rows_per_w // CHUNK

    mesh = plsc.VectorSubcoreMesh(core_axis_name="c", subcore_axis_name="s")

    @functools.partial(
        pl.kernel,
        out_type=jax.ShapeDtypeStruct((B, length, D), table.dtype),
        mesh=mesh,
        scratch_types=[
            pltpu.VMEM((CHUNK, D), table.dtype),
            pltpu.VMEM((CHUNK, D), table.dtype),
            pltpu.SemaphoreType.DMA,
            pltpu.SemaphoreType.DMA,
        ],
    )
    def sc_copy(table_hbm, out_hbm, buf0, buf1, lsem, ssem):
        wid = lax.axis_index("s") * _NC + lax.axis_index("c")
        base = wid * rows_per_w
        bufs = [buf0, buf1]
        loads = [None, None]
        stores = [[], []]

        def start_load(c):
            off = base + c * CHUNK
            return pltpu.async_copy(
                table_hbm.at[pl.ds(off, CHUNK)], bufs[c % 2], lsem)

        loads[0] = start_load(0)
        for c in range(n_chunks):
            k = c % 2
            nk = (c + 1) % 2
            for h in stores[nk]:
                h.wait()
            stores[nk] = []
            if c + 1 < n_chunks:
                loads[nk] = start_load(c + 1)
            loads[k].wait()
            off = base + c * CHUNK
            stores[k] = [
                pltpu.async_copy(
                    bufs[k], out_hbm.at[b, pl.ds(off, CHUNK)], ssem)
                for b in range(B)
            ]
        for h in stores[0] + stores[1]:
            h.wait()

    return sc_copy(table)


def _copy_body(t_ref, o_ref):
    o_ref[0] = t_ref[...]


def _kernel_tc(x, table):
    B, length, _ = x.shape
    _, D = table.shape
    BLK = 512
    out = pl.pallas_call(
        _copy_body,
        grid=(length // BLK, B),
        in_specs=[pl.BlockSpec((BLK, D), lambda i, b: (i, 0))],
        out_specs=pl.BlockSpec((1, BLK, D), lambda i, b: (b, i, 0)),
        out_shape=jax.ShapeDtypeStruct((B, length, D), table.dtype),
    )(table)
    return out


# speedup vs baseline: 50.6477x; 1.0163x over previous
import functools

import jax
import jax.numpy as jnp
from jax import lax
from jax.experimental import pallas as pl
from jax.experimental.pallas import tpu as pltpu
from jax.experimental.pallas import tpu_sc as plsc

_NC = 2
_NS = 16
_NW = _NC * _NS


def kernel(x, table):
    B, length, _ = x.shape
    V, D = table.shape
    rows_per_w = length // _NW
    CHUNK = 64
    n_chunks = rows_per_w // CHUNK

    mesh = plsc.VectorSubcoreMesh(core_axis_name="c", subcore_axis_name="s")

    @functools.partial(
        pl.kernel,
        out_type=jax.ShapeDtypeStruct((B, length, D), table.dtype),
        mesh=mesh,
        scratch_types=[
            pltpu.VMEM((CHUNK, D), table.dtype),
            pltpu.SemaphoreType.DMA,
        ],
    )
    def sc_copy(table_hbm, out_hbm, buf, ssem):
        wid = lax.axis_index("s") * _NC + lax.axis_index("c")
        base = wid * rows_per_w
        for c in range(n_chunks):
            off = base + c * CHUNK
            pltpu.sync_copy(table_hbm.at[pl.ds(off, CHUNK)], buf)
            stores = [
                pltpu.async_copy(
                    buf, out_hbm.at[b, pl.ds(off, CHUNK)], ssem)
                for b in range(B)
            ]
            for h in stores:
                h.wait()

    return sc_copy(table)


def _copy_body(t_ref, o_ref):
    o_ref[0] = t_ref[...]


def _kernel_tc(x, table):
    B, length, _ = x.shape
    _, D = table.shape
    BLK = 512
    out = pl.pallas_call(
        _copy_body,
        grid=(length // BLK, B),
        in_specs=[pl.BlockSpec((BLK, D), lambda i, b: (i, 0))],
        out_specs=pl.BlockSpec((1, BLK, D), lambda i, b: (b, i, 0)),
        out_shape=jax.ShapeDtypeStruct((B, length, D), table.dtype),
    )(table)
    return out
